# baseline (speedup 1.0000x reference)
import functools

import jax
import jax.numpy as jnp
from jax import lax
from jax.experimental import pallas as pl
from jax.experimental.pallas import tpu as pltpu
from jax.experimental.pallas import tpu_sc as plsc

N, E, F, H, G, C = 10000, 320000, 128, 64, 16, 10
F2 = 2 * H

NC, NS = 2, 16
NTILES = NC * NS
EPT = E // NTILES
NCHUNK = 80
EPTP = NCHUNK * 128
ROWS_PER_TILE = 632
NPAD = NS * ROWS_PER_TILE
PAD_SRC = N
PAD_DST = NPAD - 1


def _deg_body(dst_hbm, out_hbm, idx_v, deg_v):
    c = lax.axis_index("c")
    s = lax.axis_index("s")
    g = c * NS + s
    zero16 = jnp.zeros((16,), jnp.float32)
    one16 = jnp.ones((16,), jnp.float32)

    def zbody(i, carry):
        deg_v[pl.ds(i * 16, 16)] = zero16
        return carry

    lax.fori_loop(0, NPAD // 16, zbody, 0)

    pltpu.sync_copy(dst_hbm.at[g], idx_v)

    def body(i, carry):
        idx = idx_v[pl.ds(i * 16, 16)]
        plsc.addupdate_scatter(deg_v, [idx], one16)
        return carry

    lax.fori_loop(0, EPTP // 16, body, 0)
    pltpu.sync_copy(deg_v, out_hbm.at[g])


def _zero_acc_slab(zeros_hbm, rows, acc, base):
    pltpu.sync_copy(zeros_hbm, rows.at[0, pl.ds(0, 128)])
    for k in range(5):
        sz = 128 if k < 4 else ROWS_PER_TILE - 4 * 128
        pltpu.sync_copy(rows.at[0, pl.ds(0, sz)],
                        acc.at[pl.ds(base + k * 128, sz)])


def _write_acc_slab(rows, acc, out_hbm, c, base):
    for k in range(5):
        sz = 128 if k < 4 else ROWS_PER_TILE - 4 * 128
        pltpu.sync_copy(acc.at[pl.ds(base + k * 128, sz)],
                        rows.at[0, pl.ds(0, sz)])
        pltpu.sync_copy(rows.at[0, pl.ds(0, sz)],
                        out_hbm.at[c, pl.ds(base + k * 128, sz)])


NBUF = 2


def _agg_pipeline(tbls, sidx, didx, rows, acc, gsems, ssems, nops):
    for t in range(NBUF):
        pltpu.async_copy(tbls[t].at[sidx.at[t]], rows.at[t], gsems[t])

    def rnd(jj, carry):
        bc = jj * NBUF
        for t in range(NBUF):
            pltpu.make_async_copy(
                tbls[t].at[sidx.at[bc + t]], rows.at[t], gsems[t]).wait()
            pltpu.async_copy(rows.at[t], acc.at[didx.at[bc + t]], ssems[t],
                             add=True)
        for t in range(NBUF):
            pltpu.make_async_copy(
                rows.at[t], acc.at[didx.at[bc + t]], ssems[t]).wait()

            @pl.when(bc + NBUF + t < nops)
            def _():
                pltpu.async_copy(
                    tbls[t].at[sidx.at[bc + NBUF + t]], rows.at[t], gsems[t])
        return carry

    lax.fori_loop(0, nops // NBUF, rnd, 0)


def _load_table_slab(y_tbl_hbm, ytab, base):
    for k in range(5):
        sz = 128 if k < 4 else ROWS_PER_TILE - 4 * 128
        pltpu.sync_copy(y_tbl_hbm.at[pl.ds(base + k * 128, sz)],
                        ytab.at[pl.ds(base + k * 128, sz)])


def _agg_fs_body(y_hbm, src_hbm, dst_hbm, zeros_hbm, out_hbm, sidx, didx,
                 rows, acc, ytab, *sems):
    c = lax.axis_index("c")
    s = lax.axis_index("s")
    base = s * ROWS_PER_TILE
    _zero_acc_slab(zeros_hbm, rows, acc, base)
    _load_table_slab(y_hbm.at[c], ytab, base)
    plsc.subcore_barrier()
    for piece in range(4):
        pltpu.sync_copy(src_hbm.at[s, piece], sidx)
        pltpu.sync_copy(dst_hbm.at[s, piece], didx)
        _agg_pipeline((ytab, y_hbm.at[c]), sidx, didx, rows, acc,
                      sems[:NBUF], sems[NBUF:], NCHUNK // 4)
    plsc.subcore_barrier()
    _write_acc_slab(rows, acc, out_hbm, c, base)


def _agg_es_body(y_hbm, src_hbm, dst_hbm, zeros_hbm, out_hbm, sidx, didx,
                 rows, acc, ytab, *sems):
    c = lax.axis_index("c")
    s = lax.axis_index("s")
    g = c * NS + s
    base = s * ROWS_PER_TILE
    _zero_acc_slab(zeros_hbm, rows, acc, base)
    _load_table_slab(y_hbm, ytab, base)
    plsc.subcore_barrier()
    for piece in range(4):
        pltpu.sync_copy(src_hbm.at[g, piece], sidx)
        pltpu.sync_copy(dst_hbm.at[g, piece], didx)
        _agg_pipeline((ytab, y_hbm), sidx, didx, rows, acc,
                      sems[:NBUF], sems[NBUF:], NCHUNK // 8)
    plsc.subcore_barrier()
    _write_acc_slab(rows, acc, out_hbm, c, base)


@functools.lru_cache(maxsize=None)
def _sc_kernels():
    mesh = plsc.VectorSubcoreMesh(
        core_axis_name="c", subcore_axis_name="s",
        num_cores=NC, num_subcores=NS)
    params = pltpu.CompilerParams(needs_layout_passes=False,
                                  use_tc_tiling_on_sc=False)
    deg = pl.kernel(
        _deg_body,
        out_type=jax.ShapeDtypeStruct((NTILES, NPAD), jnp.float32),
        mesh=mesh,
        compiler_params=params,
        scratch_types=[
            pltpu.VMEM((EPTP,), jnp.int32),
            pltpu.VMEM((NPAD,), jnp.float32),
        ],
    )

    def make_agg(body, nchunk):
        return pl.kernel(
            body,
            out_type=jax.ShapeDtypeStruct((NC, NPAD, H), jnp.float32),
            mesh=mesh,
            compiler_params=params,
            scratch_types=[
                pltpu.VMEM((nchunk // 8, 256), jnp.int32),
                pltpu.VMEM((nchunk // 8, 256), jnp.int32),
                pltpu.VMEM((NBUF, 256, H), jnp.float32),
                pltpu.VMEM_SHARED((NPAD, H), jnp.float32),
                pltpu.VMEM_SHARED((NPAD, H), jnp.float32),
            ] + [pltpu.SemaphoreType.DMA] * (2 * NBUF),
        )

    return (deg, make_agg(_agg_fs_body, 2 * NCHUNK),
            make_agg(_agg_es_body, NCHUNK))


def _tc1_body(xp_ref, w1_ref, degp_ref, y1_ref, dinv_ref):
    deg = jnp.sum(degp_ref[...], axis=0) + 1.0
    dinv = lax.rsqrt(jnp.maximum(deg, 1.0))
    xw = jnp.dot(xp_ref[...], w1_ref[...], preferred_element_type=jnp.float32)
    y = xw * dinv[:, None]
    y1_ref[0] = y[:, :H]
    y1_ref[1] = y[:, H:]
    dinv_ref[...] = dinv


def _tc2_body(sp_ref, y1_ref, dinv_ref, b1_ref, w2_ref, y2_ref):
    dinv = dinv_ref[...]
    s_full = jnp.concatenate(
        [sp_ref[0] + y1_ref[0], sp_ref[1] + y1_ref[1]], axis=1)
    agg = s_full * dinv[:, None] + b1_ref[...]
    h = jnp.maximum(agg, 0.0)
    xw2 = jnp.dot(h, w2_ref[...], preferred_element_type=jnp.float32)
    y2_ref[...] = xw2 * dinv[:, None]


def _tc3_body(sp_ref, y2_ref, dinv_ref, b2_ref, batch_ref, wx_ref, bx_ref,
              wfc_ref, bfc_ref, out_ref):
    dinv = dinv_ref[...]
    agg = (sp_ref[0] + sp_ref[1] + y2_ref[...]) * dinv[:, None] + b2_ref[...]
    h2 = jnp.maximum(agg, 0.0)
    onehot = (batch_ref[...][:, None]
              == lax.broadcasted_iota(jnp.int32, (NPAD, G), 1)
              ).astype(jnp.float32)
    pooled = lax.dot_general(onehot, h2, (((0,), (0,)), ((), ())),
                             preferred_element_type=jnp.float32)
    t = jnp.dot(pooled, wx_ref[...],
                preferred_element_type=jnp.float32) + bx_ref[...]
    t = jnp.dot(t, wfc_ref[...],
                preferred_element_type=jnp.float32) + bfc_ref[...]
    m = jnp.max(t, axis=1, keepdims=True)
    lse = jnp.log(jnp.sum(jnp.exp(t - m), axis=1, keepdims=True)) + m
    out_ref[...] = t - lse


def kernel(x, edge_index, image_features, batch, W1, b1, W2, b2,
           W_x0, b_x0, W_x, b_x, W_fc, b_fc):
    del image_features, W_x0, b_x0

    src = edge_index[0].astype(jnp.int32)
    dst = edge_index[1].astype(jnp.int32)

    pad_w = EPTP - EPT
    src2 = jnp.concatenate(
        [src.reshape(NTILES, EPT),
         jnp.full((NTILES, pad_w), PAD_SRC, jnp.int32)], axis=1)
    dst2 = jnp.concatenate(
        [dst.reshape(NTILES, EPT),
         jnp.full((NTILES, pad_w), PAD_DST, jnp.int32)], axis=1)
    src_fs = src2.reshape(NS, 4, NCHUNK // 4, 256)
    dst_fs = dst2.reshape(NS, 4, NCHUNK // 4, 256)
    src_es = src2.reshape(NTILES, 4, NCHUNK // 8, 256)
    dst_es = dst2.reshape(NTILES, 4, NCHUNK // 8, 256)

    xp = jnp.zeros((NPAD, F), x.dtype).at[:N].set(x)
    batchp = jnp.concatenate(
        [batch.astype(jnp.int32), jnp.full((NPAD - N,), G, jnp.int32)])
    zeros_h = jnp.zeros((128, H), jnp.float32)

    deg_kernel, agg_fs, agg_es = _sc_kernels()

    degp = deg_kernel(dst2)

    y1, dinv = pl.pallas_call(
        _tc1_body,
        out_shape=[jax.ShapeDtypeStruct((NC, NPAD, H), jnp.float32),
                   jax.ShapeDtypeStruct((NPAD,), jnp.float32)],
    )(xp, W1, degp)

    s1 = agg_fs(y1, src_fs, dst_fs, zeros_h)

    y2 = pl.pallas_call(
        _tc2_body,
        out_shape=jax.ShapeDtypeStruct((NPAD, H), jnp.float32),
    )(s1, y1, dinv, b1, W2)

    s2 = agg_es(y2, src_es, dst_es, zeros_h)

    out = pl.pallas_call(
        _tc3_body,
        out_shape=jax.ShapeDtypeStruct((G, C), jnp.float32),
    )(s2, y2, dinv, b2, batchp, W_x, b_x, W_fc, b_fc)
    return out


# speedup vs baseline: 1.6051x; 1.6051x over previous
import functools

import jax
import jax.numpy as jnp
from jax import lax
from jax.experimental import pallas as pl
from jax.experimental.pallas import tpu as pltpu
from jax.experimental.pallas import tpu_sc as plsc

N, E, F, H, G, C = 10000, 320000, 128, 64, 16, 10
F2 = 2 * H

NC, NS = 2, 16
NTILES = NC * NS
EPT = E // NTILES
NCHUNK = 80
EPTP = NCHUNK * 128
ROWS_PER_TILE = 632
NPAD = NS * ROWS_PER_TILE
PAD_SRC = N
PAD_DST = NPAD - 1


def _deg_body(dst_hbm, out_hbm, idx_v, deg_v):
    c = lax.axis_index("c")
    s = lax.axis_index("s")
    g = c * NS + s
    zero16 = jnp.zeros((16,), jnp.float32)
    one16 = jnp.ones((16,), jnp.float32)

    def zbody(i, carry):
        deg_v[pl.ds(i * 16, 16)] = zero16
        return carry

    lax.fori_loop(0, NPAD // 16, zbody, 0)

    pltpu.sync_copy(dst_hbm.at[g], idx_v)

    def body(i, carry):
        idx = idx_v[pl.ds(i * 16, 16)]
        plsc.addupdate_scatter(deg_v, [idx], one16)
        return carry

    lax.fori_loop(0, EPTP // 16, body, 0)
    pltpu.sync_copy(deg_v, out_hbm.at[g])


def _zero_acc_slab(zeros_hbm, rows, acc, base):
    pltpu.sync_copy(zeros_hbm, rows.at[0, pl.ds(0, 128)])
    for k in range(5):
        sz = 128 if k < 4 else ROWS_PER_TILE - 4 * 128
        pltpu.sync_copy(rows.at[0, pl.ds(0, sz)],
                        acc.at[pl.ds(base + k * 128, sz)])


def _write_acc_slab(rows, acc, out_hbm, c, base):
    for k in range(5):
        sz = 128 if k < 4 else ROWS_PER_TILE - 4 * 128
        pltpu.sync_copy(acc.at[pl.ds(base + k * 128, sz)],
                        rows.at[0, pl.ds(0, sz)])
        pltpu.sync_copy(rows.at[0, pl.ds(0, sz)],
                        out_hbm.at[c, pl.ds(base + k * 128, sz)])


NBUF = 4


def _agg_pipeline(y_tbl, sidx, didx, rows, acc, gsems, ssems, nops):
    for t in range(NBUF):
        pltpu.async_copy(y_tbl.at[sidx.at[t]], rows.at[t], gsems[t])

    def rnd(jj, carry):
        bc = jj * NBUF
        for t in range(NBUF):
            pltpu.make_async_copy(
                y_tbl.at[sidx.at[bc + t]], rows.at[t], gsems[t]).wait()
            pltpu.async_copy(rows.at[t], acc.at[didx.at[bc + t]], ssems[t],
                             add=True)
        for t in range(NBUF):
            pltpu.make_async_copy(
                rows.at[t], acc.at[didx.at[bc + t]], ssems[t]).wait()

            @pl.when(bc + NBUF + t < nops)
            def _():
                pltpu.async_copy(
                    y_tbl.at[sidx.at[bc + NBUF + t]], rows.at[t], gsems[t])
        return carry

    lax.fori_loop(0, nops // NBUF, rnd, 0)


def _load_table_slab(y_tbl_hbm, ytab, base):
    for k in range(5):
        sz = 128 if k < 4 else ROWS_PER_TILE - 4 * 128
        pltpu.sync_copy(y_tbl_hbm.at[pl.ds(base + k * 128, sz)],
                        ytab.at[pl.ds(base + k * 128, sz)])


def _agg_fs_body(y_hbm, src_hbm, dst_hbm, zeros_hbm, out_hbm, sidx, didx,
                 rows, acc, ytab, *sems):
    c = lax.axis_index("c")
    s = lax.axis_index("s")
    base = s * ROWS_PER_TILE
    _zero_acc_slab(zeros_hbm, rows, acc, base)
    _load_table_slab(y_hbm.at[c], ytab, base)
    plsc.subcore_barrier()
    for piece in range(4):
        pltpu.sync_copy(src_hbm.at[s, piece], sidx)
        pltpu.sync_copy(dst_hbm.at[s, piece], didx)
        _agg_pipeline(ytab, sidx, didx, rows, acc,
                      sems[:NBUF], sems[NBUF:], NCHUNK // 2)
    plsc.subcore_barrier()
    _write_acc_slab(rows, acc, out_hbm, c, base)


def _agg_es_body(y_hbm, src_hbm, dst_hbm, zeros_hbm, out_hbm, sidx, didx,
                 rows, acc, ytab, *sems):
    c = lax.axis_index("c")
    s = lax.axis_index("s")
    g = c * NS + s
    base = s * ROWS_PER_TILE
    _zero_acc_slab(zeros_hbm, rows, acc, base)
    _load_table_slab(y_hbm, ytab, base)
    plsc.subcore_barrier()
    for piece in range(4):
        pltpu.sync_copy(src_hbm.at[g, piece], sidx)
        pltpu.sync_copy(dst_hbm.at[g, piece], didx)
        _agg_pipeline(ytab, sidx, didx, rows, acc,
                      sems[:NBUF], sems[NBUF:], NCHUNK // 4)
    plsc.subcore_barrier()
    _write_acc_slab(rows, acc, out_hbm, c, base)


@functools.lru_cache(maxsize=None)
def _sc_kernels():
    mesh = plsc.VectorSubcoreMesh(
        core_axis_name="c", subcore_axis_name="s",
        num_cores=NC, num_subcores=NS)
    params = pltpu.CompilerParams(needs_layout_passes=False,
                                  use_tc_tiling_on_sc=False)
    deg = pl.kernel(
        _deg_body,
        out_type=jax.ShapeDtypeStruct((NTILES, NPAD), jnp.float32),
        mesh=mesh,
        compiler_params=params,
        scratch_types=[
            pltpu.VMEM((EPTP,), jnp.int32),
            pltpu.VMEM((NPAD,), jnp.float32),
        ],
    )

    def make_agg(body, nchunk):
        return pl.kernel(
            body,
            out_type=jax.ShapeDtypeStruct((NC, NPAD, H), jnp.float32),
            mesh=mesh,
            compiler_params=params,
            scratch_types=[
                pltpu.VMEM((nchunk // 4, 128), jnp.int32),
                pltpu.VMEM((nchunk // 4, 128), jnp.int32),
                pltpu.VMEM((NBUF, 128, H), jnp.float32),
                pltpu.VMEM_SHARED((NPAD, H), jnp.float32),
                pltpu.VMEM_SHARED((NPAD, H), jnp.float32),
            ] + [pltpu.SemaphoreType.DMA] * (2 * NBUF),
        )

    return (deg, make_agg(_agg_fs_body, 2 * NCHUNK),
            make_agg(_agg_es_body, NCHUNK))


def _tc1_body(xp_ref, w1_ref, degp_ref, y1_ref, dinv_ref):
    deg = jnp.sum(degp_ref[...], axis=0) + 1.0
    dinv = lax.rsqrt(jnp.maximum(deg, 1.0))
    xw = jnp.dot(xp_ref[...], w1_ref[...], preferred_element_type=jnp.float32)
    y = xw * dinv[:, None]
    y1_ref[0] = y[:, :H]
    y1_ref[1] = y[:, H:]
    dinv_ref[...] = dinv


def _tc2_body(sp_ref, y1_ref, dinv_ref, b1_ref, w2_ref, y2_ref):
    dinv = dinv_ref[...]
    s_full = jnp.concatenate(
        [sp_ref[0] + y1_ref[0], sp_ref[1] + y1_ref[1]], axis=1)
    agg = s_full * dinv[:, None] + b1_ref[...]
    h = jnp.maximum(agg, 0.0)
    xw2 = jnp.dot(h, w2_ref[...], preferred_element_type=jnp.float32)
    y2_ref[...] = xw2 * dinv[:, None]


def _tc3_body(sp_ref, y2_ref, dinv_ref, b2_ref, batch_ref, wx_ref, bx_ref,
              wfc_ref, bfc_ref, out_ref):
    dinv = dinv_ref[...]
    agg = (sp_ref[0] + sp_ref[1] + y2_ref[...]) * dinv[:, None] + b2_ref[...]
    h2 = jnp.maximum(agg, 0.0)
    onehot = (batch_ref[...][:, None]
              == lax.broadcasted_iota(jnp.int32, (NPAD, G), 1)
              ).astype(jnp.float32)
    pooled = lax.dot_general(onehot, h2, (((0,), (0,)), ((), ())),
                             preferred_element_type=jnp.float32)
    t = jnp.dot(pooled, wx_ref[...],
                preferred_element_type=jnp.float32) + bx_ref[...]
    t = jnp.dot(t, wfc_ref[...],
                preferred_element_type=jnp.float32) + bfc_ref[...]
    m = jnp.max(t, axis=1, keepdims=True)
    lse = jnp.log(jnp.sum(jnp.exp(t - m), axis=1, keepdims=True)) + m
    out_ref[...] = t - lse


def kernel(x, edge_index, image_features, batch, W1, b1, W2, b2,
           W_x0, b_x0, W_x, b_x, W_fc, b_fc):
    del image_features, W_x0, b_x0

    src = edge_index[0].astype(jnp.int32)
    dst = edge_index[1].astype(jnp.int32)

    pad_w = EPTP - EPT
    src2 = jnp.concatenate(
        [src.reshape(NTILES, EPT),
         jnp.full((NTILES, pad_w), PAD_SRC, jnp.int32)], axis=1)
    dst2 = jnp.concatenate(
        [dst.reshape(NTILES, EPT),
         jnp.full((NTILES, pad_w), PAD_DST, jnp.int32)], axis=1)
    src_fs = src2.reshape(NS, 4, NCHUNK // 2, 128)
    dst_fs = dst2.reshape(NS, 4, NCHUNK // 2, 128)
    src_es = src2.reshape(NTILES, 4, NCHUNK // 4, 128)
    dst_es = dst2.reshape(NTILES, 4, NCHUNK // 4, 128)

    xp = jnp.zeros((NPAD, F), x.dtype).at[:N].set(x)
    batchp = jnp.concatenate(
        [batch.astype(jnp.int32), jnp.full((NPAD - N,), G, jnp.int32)])
    zeros_h = jnp.zeros((128, H), jnp.float32)

    deg_kernel, agg_fs, agg_es = _sc_kernels()

    degp = deg_kernel(dst2)

    y1, dinv = pl.pallas_call(
        _tc1_body,
        out_shape=[jax.ShapeDtypeStruct((NC, NPAD, H), jnp.float32),
                   jax.ShapeDtypeStruct((NPAD,), jnp.float32)],
    )(xp, W1, degp)

    s1 = agg_fs(y1, src_fs, dst_fs, zeros_h)

    y2 = pl.pallas_call(
        _tc2_body,
        out_shape=jax.ShapeDtypeStruct((NPAD, H), jnp.float32),
    )(s1, y1, dinv, b1, W2)

    s2 = agg_es(y2, src_es, dst_es, zeros_h)

    out = pl.pallas_call(
        _tc3_body,
        out_shape=jax.ShapeDtypeStruct((G, C), jnp.float32),
    )(s2, y2, dinv, b2, batchp, W_x, b_x, W_fc, b_fc)
    return out
